# baseline (speedup 1.0000x reference)
import functools

import jax
import jax.numpy as jnp
from jax import lax
from jax.experimental import pallas as pl
from jax.experimental.pallas import tpu as pltpu
from jax.experimental.pallas import tpu_sc as plsc

N = 100000
J = 16
L = 16
NC = 2
NS = 16
W = NC * NS
GROUPS = N // L
GP_BASE = GROUPS // W
GP_EXTRA = GROUPS - GP_BASE * W
MAXG = GP_BASE + 1
MAXN = MAXG * L
G = 1024
ROW = 32
ACC = G * ROW
ACC_PER_SUB = ACC // NS
PRE_GRID = 4
PRE_B = N // PRE_GRID
PRE_R = 800 // PRE_GRID


def _pre_kernel(x_ref, y_ref, f_ref, p_ref):
    x = x_ref[...]
    f_ref[...] = jnp.maximum(x, 0.0) + jnp.log1p(jnp.exp(-jnp.abs(x)))
    y = y_ref[...]
    wts = (jnp.int32(1) << lax.broadcasted_iota(jnp.int32, (1, J), 1))
    p = jnp.sum(y * wts, axis=1)
    p_ref[...] = p.reshape(p_ref.shape)


def _fin_kernel(acc_ref, b_ref, o_ref):
    a = acc_ref[0] + acc_ref[1]
    s = a[:, 0:16]
    mx = jnp.max(s, axis=1, keepdims=True)
    f_sum = a[:, 16:17]
    cnt = a[:, 17:18]
    rows = lax.broadcasted_iota(jnp.int32, (G, 1), 0)
    ng = jnp.max(b_ref[...]) + 1
    val = jnp.where((cnt > 0.0) & (rows < ng),
                    (f_sum - mx) / jnp.maximum(cnt, 1.0), 0.0)
    o_ref[...] = jnp.full((1, 1), jnp.sum(val) / ng.astype(jnp.float32))


def _bcast_lane(vec, j):
    idx = jnp.full((L,), j, jnp.int32)
    return vec.at[idx].get(mode="promise_in_bounds")


def _sc_body(x_hbm, f_hbm, b_hbm, p_hbm, out_hbm,
             x_v, f_v, g_v, p_v, stg_v, idx_v, zb_v, acc_sh, sem):
    cid = lax.axis_index("c")
    sid = lax.axis_index("s")
    wid = cid * NS + sid

    gs = GP_BASE * wid + jnp.minimum(wid, GP_EXTRA)
    ngroups = jnp.where(wid < GP_EXTRA, GP_BASE + 1, GP_BASE)
    off = jnp.minimum(gs * L, N - MAXN)
    lo = gs * L - off

    c1 = pltpu.async_copy(x_hbm.at[pl.ds(off, MAXN)], x_v, sem)
    c2 = pltpu.async_copy(f_hbm.at[pl.ds(off, MAXN)], f_v, sem)
    c3 = pltpu.async_copy(b_hbm.at[pl.ds(off, MAXN)], g_v, sem)
    c4 = pltpu.async_copy(p_hbm.at[pl.ds(off, MAXN)], p_v, sem)

    li = lax.iota(jnp.int32, L)
    zf = jnp.zeros((L,), jnp.float32)

    def zero_zb(r, _):
        zb_v[pl.ds(r * L, L)] = zf
        return 0

    lax.fori_loop(0, ACC_PER_SUB // L, zero_zb, 0)

    pltpu.sync_copy(zb_v, acc_sh.at[pl.ds(sid * ACC_PER_SUB, ACC_PER_SUB)])
    c1.wait()
    c2.wait()
    c3.wait()
    c4.wait()
    plsc.subcore_barrier()

    lane0 = li == 0
    lane1 = li == 1
    zeros = jnp.zeros((L,), jnp.float32)
    ones = jnp.ones((L,), jnp.float32)
    one_i = jnp.ones((L,), jnp.int32)

    gv0 = g_v[pl.ds(lo, L)].astype(jnp.float32)
    prev_g0 = jnp.sum(jnp.where(lane0, gv0, jnp.zeros_like(gv0)))

    def do_flush(pg, acc_s, acc_fc):
        base = pg.astype(jnp.int32) * ROW
        stg_v[pl.ds(0, L)] = acc_s
        stg_v[pl.ds(L, L)] = acc_fc
        idx_v[pl.ds(0, L)] = base + li
        idx_v[pl.ds(L, L)] = base + L + li
        pltpu.sync_copy(stg_v, acc_sh.at[idx_v], add=True)

    def yrow(pv, j):
        pb = _bcast_lane(pv, j)
        return ((pb >> li) & one_i).astype(jnp.float32)

    def body(i, carry):
        prev_g, acc_s, acc_fc = carry
        nb = lo + i * L
        gv = g_v[pl.ds(nb, L)].astype(jnp.float32)
        xv = x_v[pl.ds(nb, L)]
        fv = f_v[pl.ds(nb, L)]
        pv = p_v[pl.ds(nb, L)]
        gmn = jnp.min(gv)
        gmx = jnp.max(gv)

        def fast(args):
            prev_g, acc_s, acc_fc = args
            changed = gmn != prev_g

            @pl.when(changed)
            def _():
                do_flush(prev_g, acc_s, acc_fc)

            pvb = jnp.full((L,), changed)
            acc_s = jnp.where(pvb, zeros, acc_s)
            acc_fc = jnp.where(pvb, zeros, acc_fc)
            acc_b = zeros
            for j in range(0, L, 2):
                acc_s = acc_s + _bcast_lane(xv, j) * yrow(pv, j)
                acc_b = acc_b + _bcast_lane(xv, j + 1) * yrow(pv, j + 1)
            acc_s = acc_s + acc_b
            sf = jnp.sum(fv)
            acc_fc = acc_fc + jnp.where(lane0, jnp.full((L,), sf), zeros) \
                            + jnp.where(lane1, ones * float(L), zeros)
            return gmn, acc_s, acc_fc

        def slow(args):
            prev_g, acc_s, acc_fc = args
            for j in range(L):
                gb = _bcast_lane(gv, j)
                gj = jnp.sum(jnp.where(lane0, gb, zeros))
                changed = gj != prev_g

                @pl.when(changed)
                def _(pgx=prev_g, asx=acc_s, afx=acc_fc):
                    do_flush(pgx, asx, afx)

                pvb = jnp.full((L,), changed)
                acc_s = jnp.where(pvb, zeros, acc_s)
                acc_fc = jnp.where(pvb, zeros, acc_fc)
                acc_s = acc_s + _bcast_lane(xv, j) * yrow(pv, j)
                fb = _bcast_lane(fv, j)
                acc_fc = acc_fc + jnp.where(lane0, fb, zeros) \
                                + jnp.where(lane1, ones, zeros)
                prev_g = jnp.where(changed, gj, prev_g)
            return prev_g, acc_s, acc_fc

        return lax.cond(gmn == gmx, fast, slow,
                        (prev_g, acc_s, acc_fc))

    prev_g, acc_s, acc_fc = lax.fori_loop(
        0, ngroups, body, (prev_g0, zeros, zeros))
    do_flush(prev_g, acc_s, acc_fc)
    plsc.subcore_barrier()

    pltpu.sync_copy(acc_sh.at[pl.ds(sid * ACC_PER_SUB, ACC_PER_SUB)],
                    out_hbm.at[cid, pl.ds(sid * ACC_PER_SUB, ACC_PER_SUB)])


@functools.partial(
    pl.kernel,
    out_type=jax.ShapeDtypeStruct((NC, ACC), jnp.float32),
    mesh=plsc.VectorSubcoreMesh(core_axis_name="c", subcore_axis_name="s"),
    compiler_params=pltpu.CompilerParams(needs_layout_passes=False),
    scratch_types=[
        pltpu.VMEM((MAXN,), jnp.float32),
        pltpu.VMEM((MAXN,), jnp.float32),
        pltpu.VMEM((MAXN,), jnp.int32),
        pltpu.VMEM((MAXN,), jnp.int32),
        pltpu.VMEM((2 * L,), jnp.float32),
        pltpu.VMEM((2 * L,), jnp.int32),
        pltpu.VMEM((ACC_PER_SUB,), jnp.float32),
        pltpu.VMEM_SHARED((ACC,), jnp.float32),
        pltpu.SemaphoreType.DMA,
    ],
)
def _sc_call(x_hbm, f_hbm, b_hbm, p_hbm, out_hbm,
             x_v, f_v, g_v, p_v, stg_v, idx_v, zb_v, acc_sh, sem):
    _sc_body(x_hbm, f_hbm, b_hbm, p_hbm, out_hbm,
             x_v, f_v, g_v, p_v, stg_v, idx_v, zb_v, acc_sh, sem)


def kernel(logits, y, batch):
    x = logits.reshape(N).astype(jnp.float32)
    yi = y.astype(jnp.int32)
    bi = batch.astype(jnp.int32)

    f2d, p2d = pl.pallas_call(
        _pre_kernel,
        grid=(PRE_GRID,),
        in_specs=[
            pl.BlockSpec((PRE_R, 125), lambda i: (i, 0)),
            pl.BlockSpec((PRE_B, J), lambda i: (i, 0)),
        ],
        out_specs=[
            pl.BlockSpec((PRE_R, 125), lambda i: (i, 0)),
            pl.BlockSpec((PRE_R, 125), lambda i: (i, 0)),
        ],
        out_shape=[
            jax.ShapeDtypeStruct((800, 125), jnp.float32),
            jax.ShapeDtypeStruct((800, 125), jnp.int32),
        ],
    )(x.reshape(800, 125), yi)
    f = f2d.reshape(N)
    p = p2d.reshape(N)

    acc = _sc_call(x, f, bi, p).reshape(NC, G, ROW)

    res = pl.pallas_call(
        _fin_kernel,
        out_shape=jax.ShapeDtypeStruct((1, 1), jnp.float32),
    )(acc, bi.reshape(800, 125))
    return res[0, 0]


# speedup vs baseline: 1.0900x; 1.0900x over previous
import functools

import jax
import jax.numpy as jnp
from jax import lax
from jax.experimental import pallas as pl
from jax.experimental.pallas import tpu as pltpu
from jax.experimental.pallas import tpu_sc as plsc

N = 100000
J = 16
L = 16
NC = 2
NS = 16
W = NC * NS
GROUPS = N // L
GP_BASE = GROUPS // W
GP_EXTRA = GROUPS - GP_BASE * W
MAXG = GP_BASE + 1
MAXN = MAXG * L
G = 1024
ROW = 32
ACC = G * ROW
ACC_PER_SUB = ACC // NS
PACK_GRID = 10
PACK_B = N // PACK_GRID


def _f_kernel(x_ref, o_ref):
    x = x_ref[...]
    o_ref[...] = jnp.maximum(x, 0.0) + jnp.log1p(jnp.exp(-jnp.abs(x)))


def _pack_kernel(y_ref, p_ref):
    yf = y_ref[...].astype(jnp.float32)
    wts = (jnp.int32(1) << lax.broadcasted_iota(jnp.int32, (J, 1), 0))
    p = jnp.dot(yf, wts.astype(jnp.float32))
    p_ref[...] = p.astype(jnp.int32)


def _fin_kernel(acc_ref, b_ref, o_ref):
    a = acc_ref[0] + acc_ref[1]
    s = a[:, 0:16]
    mx = jnp.max(s, axis=1, keepdims=True)
    f_sum = a[:, 16:17]
    cnt = a[:, 17:18]
    rows = lax.broadcasted_iota(jnp.int32, (G, 1), 0)
    ng = jnp.max(b_ref[...]) + 1
    val = jnp.where((cnt > 0.0) & (rows < ng),
                    (f_sum - mx) / jnp.maximum(cnt, 1.0), 0.0)
    o_ref[...] = jnp.full((1, 1), jnp.sum(val) / ng.astype(jnp.float32))


def _bcast_lane(vec, j):
    idx = jnp.full((L,), j, jnp.int32)
    return vec.at[idx].get(mode="promise_in_bounds")


def _sc_body(x_hbm, f_hbm, b_hbm, p_hbm, out_hbm,
             x_v, f_v, g_v, p_v, stg_v, idx_v, zb_v, acc_sh, sem):
    cid = lax.axis_index("c")
    sid = lax.axis_index("s")
    wid = cid * NS + sid

    gs = GP_BASE * wid + jnp.minimum(wid, GP_EXTRA)
    ngroups = jnp.where(wid < GP_EXTRA, GP_BASE + 1, GP_BASE)
    off = jnp.minimum(gs * L, N - MAXN)
    lo = gs * L - off

    c1 = pltpu.async_copy(x_hbm.at[pl.ds(off, MAXN)], x_v, sem)
    c2 = pltpu.async_copy(f_hbm.at[pl.ds(off, MAXN)], f_v, sem)
    c3 = pltpu.async_copy(b_hbm.at[pl.ds(off, MAXN)], g_v, sem)
    c4 = pltpu.async_copy(p_hbm.at[pl.ds(off, MAXN)], p_v, sem)

    li = lax.iota(jnp.int32, L)
    zf = jnp.zeros((L,), jnp.float32)

    def zero_zb(r, _):
        zb_v[pl.ds(r * L, L)] = zf
        return 0

    lax.fori_loop(0, ACC_PER_SUB // L, zero_zb, 0)

    pltpu.sync_copy(zb_v, acc_sh.at[pl.ds(sid * ACC_PER_SUB, ACC_PER_SUB)])
    c1.wait()
    c2.wait()
    c3.wait()
    c4.wait()
    plsc.subcore_barrier()

    lane0 = li == 0
    lane1 = li == 1
    zeros = jnp.zeros((L,), jnp.float32)
    bitmask = jnp.int32(1) << li
    zero_i = jnp.zeros((L,), jnp.int32)

    prev_g0 = g_v[pl.ds(lo, L)][0]

    def do_flush(pg, acc_s, acc_fv, cnt):
        base = pg * ROW
        sf = jnp.sum(acc_fv)
        fc = jnp.where(lane0, jnp.full((L,), sf), zeros) \
           + jnp.where(lane1, jnp.full((L,), cnt.astype(jnp.float32)), zeros)
        stg_v[pl.ds(0, L)] = acc_s
        stg_v[pl.ds(L, L)] = fc
        idx_v[pl.ds(0, L)] = base + li
        idx_v[pl.ds(L, L)] = base + L + li
        pltpu.sync_copy(stg_v, acc_sh.at[idx_v], add=True)

    def body(i, carry):
        prev_g, acc_s, acc_fv, cnt = carry
        nb = lo + i * L
        gvi = g_v[pl.ds(nb, L)]
        xv = x_v[pl.ds(nb, L)]
        fv = f_v[pl.ds(nb, L)]
        pv = p_v[pl.ds(nb, L)]
        g_first = gvi[0]
        g_last = gvi[15]

        def fast(args):
            prev_g, acc_s, acc_fv, cnt = args
            changed = g_first != prev_g

            @pl.when(changed)
            def _():
                do_flush(prev_g, acc_s, acc_fv, cnt)

            pvb = jnp.full((L,), changed)
            acc_s = jnp.where(pvb, zeros, acc_s)
            acc_fv = jnp.where(pvb, zeros, acc_fv)
            cnt = jnp.where(changed, 0, cnt)
            acc_b = zeros
            for j in range(0, L, 2):
                pb = _bcast_lane(pv, j)
                m = (pb & bitmask) != zero_i
                xb = _bcast_lane(xv, j)
                acc_s = jnp.where(m, acc_s + xb, acc_s)
                pb2 = _bcast_lane(pv, j + 1)
                m2 = (pb2 & bitmask) != zero_i
                xb2 = _bcast_lane(xv, j + 1)
                acc_b = jnp.where(m2, acc_b + xb2, acc_b)
            acc_s = acc_s + acc_b
            acc_fv = acc_fv + fv
            cnt = cnt + L
            return g_first, acc_s, acc_fv, cnt

        def slow(args):
            prev_g, acc_s, acc_fv, cnt = args
            for j in range(L):
                gj = gvi[j]
                changed = gj != prev_g

                @pl.when(changed)
                def _(pgx=prev_g, asx=acc_s, afx=acc_fv, cnx=cnt):
                    do_flush(pgx, asx, afx, cnx)

                pvb = jnp.full((L,), changed)
                acc_s = jnp.where(pvb, zeros, acc_s)
                acc_fv = jnp.where(pvb, zeros, acc_fv)
                cnt = jnp.where(changed, 0, cnt)
                pb = _bcast_lane(pv, j)
                m = (pb & bitmask) != zero_i
                xb = _bcast_lane(xv, j)
                acc_s = jnp.where(m, acc_s + xb, acc_s)
                acc_fv = acc_fv + jnp.where(li == j, fv, zeros)
                cnt = cnt + 1
                prev_g = jnp.where(changed, gj, prev_g)
            return prev_g, acc_s, acc_fv, cnt

        return lax.cond(g_first == g_last, fast, slow,
                        (prev_g, acc_s, acc_fv, cnt))

    prev_g, acc_s, acc_fv, cnt = lax.fori_loop(
        0, ngroups, body, (prev_g0, zeros, zeros, jnp.int32(0)))
    do_flush(prev_g, acc_s, acc_fv, cnt)
    plsc.subcore_barrier()

    pltpu.sync_copy(acc_sh.at[pl.ds(sid * ACC_PER_SUB, ACC_PER_SUB)],
                    out_hbm.at[cid, pl.ds(sid * ACC_PER_SUB, ACC_PER_SUB)])


@functools.partial(
    pl.kernel,
    out_type=jax.ShapeDtypeStruct((NC, ACC), jnp.float32),
    mesh=plsc.VectorSubcoreMesh(core_axis_name="c", subcore_axis_name="s"),
    compiler_params=pltpu.CompilerParams(needs_layout_passes=False),
    scratch_types=[
        pltpu.VMEM((MAXN,), jnp.float32),
        pltpu.VMEM((MAXN,), jnp.float32),
        pltpu.VMEM((MAXN,), jnp.int32),
        pltpu.VMEM((MAXN,), jnp.int32),
        pltpu.VMEM((2 * L,), jnp.float32),
        pltpu.VMEM((2 * L,), jnp.int32),
        pltpu.VMEM((ACC_PER_SUB,), jnp.float32),
        pltpu.VMEM_SHARED((ACC,), jnp.float32),
        pltpu.SemaphoreType.DMA,
    ],
)
def _sc_call(x_hbm, f_hbm, b_hbm, p_hbm, out_hbm,
             x_v, f_v, g_v, p_v, stg_v, idx_v, zb_v, acc_sh, sem):
    _sc_body(x_hbm, f_hbm, b_hbm, p_hbm, out_hbm,
             x_v, f_v, g_v, p_v, stg_v, idx_v, zb_v, acc_sh, sem)


def kernel(logits, y, batch):
    x = logits.reshape(N).astype(jnp.float32)
    yi = y.astype(jnp.int32)
    bi = batch.astype(jnp.int32)

    f2d = pl.pallas_call(
        _f_kernel,
        out_shape=jax.ShapeDtypeStruct((800, 125), jnp.float32),
    )(x.reshape(800, 125))
    f = f2d.reshape(N)

    p2d = pl.pallas_call(
        _pack_kernel,
        grid=(PACK_GRID,),
        in_specs=[pl.BlockSpec((PACK_B, J), lambda i: (i, 0))],
        out_specs=pl.BlockSpec((PACK_B, 1), lambda i: (i, 0)),
        out_shape=jax.ShapeDtypeStruct((N, 1), jnp.int32),
    )(yi)
    p = p2d.reshape(N)

    acc = _sc_call(x, f, bi, p).reshape(NC, G, ROW)

    res = pl.pallas_call(
        _fin_kernel,
        out_shape=jax.ShapeDtypeStruct((1, 1), jnp.float32),
    )(acc, bi.reshape(800, 125))
    return res[0, 0]
